# baseline (speedup 1.0000x reference)
import functools
import jax
import jax.numpy as jnp
from jax import lax
from jax.experimental import pallas as pl
from jax.experimental.pallas import tpu as pltpu
from jax.experimental.pallas import tpu_sc as plsc

G = 128
F = 8
BOUND = 1.0
N = 2097152

NC, NS, L = 2, 16, 16
NW = NC * NS
M = N // NW
B = 2048
NBLK = M // B
CHUNK = 128
NCHUNK = B // CHUNK

_ROW_WH = 0
_ROW_WD = 24
_ROW_BC = 33
_ROW_AC = 36
_NCONST = 40


def _sc_forward(x, d, dens2, feat, consts):
    mesh = plsc.VectorSubcoreMesh(core_axis_name="c", subcore_axis_name="s")

    @functools.partial(
        pl.kernel,
        out_type=(
            jax.ShapeDtypeStruct((N,), jnp.float32),
            jax.ShapeDtypeStruct((N, 3), jnp.float32),
            jax.ShapeDtypeStruct((N, 3), jnp.float32),
        ),
        mesh=mesh,
        compiler_params=pltpu.CompilerParams(
            use_tc_tiling_on_sc=False, needs_layout_passes=False),
        scratch_types=[
            pltpu.VMEM((B, 3), jnp.float32),
            pltpu.VMEM((B, 3), jnp.float32),
            pltpu.VMEM((B,), jnp.int32),
            pltpu.VMEM((B, F), jnp.float32),
            pltpu.VMEM((B,), jnp.float32),
            pltpu.VMEM((B,), jnp.float32),
            pltpu.VMEM((B, 3), jnp.float32),
            pltpu.VMEM((B, 3), jnp.float32),
            pltpu.VMEM((_NCONST, L), jnp.float32),
            pltpu.SemaphoreType.DMA,
            pltpu.SemaphoreType.DMA,
        ],
    )
    def k(x_h, d_h, dens_h, feat_h, consts_h, sig_h, rgb_h, amb_h,
          xv, dv, idxv, featv, densv, sigv, rgbv, ambv, cv, sem_f, sem_d):
        wid = lax.axis_index("s") * NC + lax.axis_index("c")
        pltpu.sync_copy(consts_h, cv)
        iota = lax.iota(jnp.int32, L)
        col0 = jnp.zeros((L,), jnp.int32)
        col1 = jnp.full((L,), 1, jnp.int32)
        col2 = jnp.full((L,), 2, jnp.int32)

        def blk_body(blk, carry):
            base = wid * M + blk * B
            pltpu.sync_copy(x_h.at[pl.ds(base, B), :], xv)
            pltpu.sync_copy(d_h.at[pl.ds(base, B), :], dv)

            def vox(g, c):
                rows = g * L + iota
                gx = plsc.load_gather(xv, [rows, col0])
                gy = plsc.load_gather(xv, [rows, col1])
                gz = plsc.load_gather(xv, [rows, col2])

                def q(v):
                    v = (v + BOUND) * ((G - 1) / (2.0 * BOUND))
                    v = jnp.minimum(jnp.maximum(v, 0.0), float(G - 1))
                    return v.astype(jnp.int32)

                idxv[pl.ds(g * L, L)] = q(gx) * (G * G) + q(gy) * G + q(gz)
                return c

            lax.fori_loop(0, B // L, vox, 0)

            cps = []
            for ci in range(NCHUNK):
                s = pl.ds(ci * CHUNK, CHUNK)
                cps.append(pltpu.async_copy(
                    feat_h.at[idxv.at[s]], featv.at[s, :], sem_f))
                cps.append(pltpu.async_copy(
                    dens_h.at[idxv.at[s]], densv.at[s], sem_d))
            for cp in cps:
                cp.wait()

            def comp(g, c):
                rows = g * L + iota
                dens = densv[pl.ds(g * L, L)]
                sigv[pl.ds(g * L, L)] = jnp.exp(dens)
                f = [plsc.load_gather(featv, [rows, jnp.full((L,), j, jnp.int32)])
                     for j in range(F)]
                dd = [plsc.load_gather(dv, [rows, cc])
                      for cc in (col0, col1, col2)]
                for kk in range(3):
                    colk = jnp.full((L,), kk, jnp.int32)
                    acc = cv[_ROW_BC + kk]
                    for j in range(F):
                        acc = acc + f[j] * cv[_ROW_WH + j * 3 + kk]
                    for i in range(3):
                        acc = acc + dd[i] * cv[_ROW_WD + i * 3 + kk]
                    rgb = 1.0 / (1.0 + jnp.exp(-acc))
                    plsc.store_scatter(rgbv, [rows, colk], rgb)
                    amb = f[kk] + cv[_ROW_AC + kk]
                    plsc.store_scatter(ambv, [rows, colk], amb)
                return c

            lax.fori_loop(0, B // L, comp, 0)

            pltpu.sync_copy(sigv, sig_h.at[pl.ds(base, B)])
            pltpu.sync_copy(rgbv, rgb_h.at[pl.ds(base, B), :])
            pltpu.sync_copy(ambv, amb_h.at[pl.ds(base, B), :])
            return carry

        lax.fori_loop(0, NBLK, blk_body, 0)

    return k(x, d, dens2, feat, consts)


def kernel(x, d, l, R, a, c, density_grid, feature_grid, W_aud, b_aud,
           W_ind, W_rgb, b_rgb):
    aud = jnp.tanh(a @ W_aud + b_aud)
    ind = c @ W_ind
    cvec = aud[0] + ind
    geo = jnp.sum((l @ R) ** 2) * 1e-4
    Wh = W_rgb[:F]
    Wd = R @ W_rgb[F:]
    bc = b_rgb + cvec @ Wh
    ac = cvec[:3] + geo
    cflat = jnp.concatenate([
        Wh.reshape(-1), Wd.reshape(-1), bc, ac,
        jnp.zeros((_NCONST - 39,), jnp.float32)])
    consts = jnp.broadcast_to(cflat[:, None], (_NCONST, L)).astype(jnp.float32)
    return _sc_forward(x, d, density_grid, feature_grid, consts)


# speedup vs baseline: 7.1379x; 7.1379x over previous
import functools
import jax
import jax.numpy as jnp
from jax import lax
from jax.experimental import pallas as pl
from jax.experimental.pallas import tpu as pltpu
from jax.experimental.pallas import tpu_sc as plsc

G = 128
F = 8
BOUND = 1.0
N = 2097152
G3 = G ** 3

NC, NS, L = 2, 16, 16
NW = NC * NS
M = N // NW
B = 2048
NBLK = M // B
CHUNK = 128
NCHUNK = B // CHUNK

MV = G3 // NW
BV = 4096
NBV = MV // BV

_ROW_WH = 0
_ROW_WD = 24
_ROW_BC = 33
_ROW_AC = 36
_NCONST = 40

_SC_PARAMS = pltpu.CompilerParams(
    use_tc_tiling_on_sc=False, needs_layout_passes=False)
_MESH = plsc.VectorSubcoreMesh(core_axis_name="c", subcore_axis_name="s")


@functools.partial(
    pl.kernel,
    out_type=jax.ShapeDtypeStruct((G3 * F,), jnp.float32),
    mesh=_MESH,
    compiler_params=_SC_PARAMS,
    scratch_types=[pltpu.VMEM((BV,), jnp.float32) for _ in range(F)]
    + [pltpu.VMEM((BV * F,), jnp.float32)],
)
def _sc_interleave(p0, p1, p2, p3, p4, p5, p6, p7, ft_h,
                   v0, v1, v2, v3, v4, v5, v6, v7, ov):
    planes = (p0, p1, p2, p3, p4, p5, p6, p7)
    bufs = (v0, v1, v2, v3, v4, v5, v6, v7)
    wid = lax.axis_index("s") * NC + lax.axis_index("c")
    iota = lax.iota(jnp.int32, L)

    def blk_body(blk, carry):
        base = wid * MV + blk * BV
        for j in range(F):
            pltpu.sync_copy(planes[j].at[pl.ds(base, BV)], bufs[j])

        def ilv(g, c):
            rows = (g * L + iota) * F
            for j in range(F):
                vj = bufs[j][pl.ds(g * L, L)]
                plsc.store_scatter(ov, [rows + j], vj)
            return c

        lax.fori_loop(0, BV // L, ilv, 0)
        pltpu.sync_copy(ov, ft_h.at[pl.ds(base * F, BV * F)])
        return carry

    lax.fori_loop(0, NBV, blk_body, 0)


@functools.partial(
    pl.kernel,
    out_type=tuple(jax.ShapeDtypeStruct((N,), jnp.float32) for _ in range(7)),
    mesh=_MESH,
    compiler_params=_SC_PARAMS,
    scratch_types=[pltpu.VMEM((B,), jnp.float32) for _ in range(6)]
    + [pltpu.VMEM((B,), jnp.int32), pltpu.VMEM((B, F), jnp.float32)]
    + [pltpu.VMEM((B,), jnp.float32) for _ in range(8)]
    + [pltpu.VMEM((_NCONST * L,), jnp.float32),
       pltpu.SemaphoreType.DMA, pltpu.SemaphoreType.DMA],
)
def _sc_field(x0_h, x1_h, x2_h, d0_h, d1_h, d2_h, dens_h, ft_h, c_h,
              sig_h, r_h, g_h, b_h, a0_h, a1_h, a2_h,
              xv0, xv1, xv2, dv0, dv1, dv2, idxv, featv,
              densv, sigv, rv, gv, bv, av0, av1, av2,
              cv, sem_f, sem_d):
    wid = lax.axis_index("s") * NC + lax.axis_index("c")
    pltpu.sync_copy(c_h, cv)
    iota = lax.iota(jnp.int32, L)

    def crow(r):
        return cv[pl.ds(r * L, L)]

    def blk_body(blk, carry):
        base = wid * M + blk * B
        for src, dst in ((x0_h, xv0), (x1_h, xv1), (x2_h, xv2),
                         (d0_h, dv0), (d1_h, dv1), (d2_h, dv2)):
            pltpu.sync_copy(src.at[pl.ds(base, B)], dst)

        def vox(g, c):
            s = pl.ds(g * L, L)

            def q(v):
                v = (v + BOUND) * ((G - 1) / (2.0 * BOUND))
                v = jnp.minimum(jnp.maximum(v, 0.0), float(G - 1))
                return v.astype(jnp.int32)

            idxv[s] = q(xv0[s]) * (G * G) + q(xv1[s]) * G + q(xv2[s])
            return c

        lax.fori_loop(0, B // L, vox, 0)

        cps = []
        for ci in range(NCHUNK):
            s = pl.ds(ci * CHUNK, CHUNK)
            cps.append(pltpu.async_copy(
                ft_h.at[idxv.at[s]], featv.at[s, :], sem_f))
            cps.append(pltpu.async_copy(
                dens_h.at[idxv.at[s]], densv.at[s], sem_d))
        for cp in cps:
            cp.wait()

        def comp(g, c):
            s = pl.ds(g * L, L)
            rows = g * L + iota
            sigv[s] = jnp.exp(densv[s])
            f = [plsc.load_gather(featv, [rows, jnp.full((L,), j, jnp.int32)])
                 for j in range(F)]
            dd = (dv0[s], dv1[s], dv2[s])
            outs = (rv, gv, bv)
            ambs = (av0, av1, av2)
            for kk in range(3):
                acc = crow(_ROW_BC + kk)
                for j in range(F):
                    acc = acc + f[j] * crow(_ROW_WH + j * 3 + kk)
                for i in range(3):
                    acc = acc + dd[i] * crow(_ROW_WD + i * 3 + kk)
                outs[kk][s] = 1.0 / (1.0 + jnp.exp(-acc))
                ambs[kk][s] = f[kk] + crow(_ROW_AC + kk)
            return c

        lax.fori_loop(0, B // L, comp, 0)

        for src, dst in ((sigv, sig_h), (rv, r_h), (gv, g_h), (bv, b_h),
                         (av0, a0_h), (av1, a1_h), (av2, a2_h)):
            pltpu.sync_copy(src, dst.at[pl.ds(base, B)])
        return carry

    lax.fori_loop(0, NBLK, blk_body, 0)


def kernel(x, d, l, R, a, c, density_grid, feature_grid, W_aud, b_aud,
           W_ind, W_rgb, b_rgb):
    aud = jnp.tanh(a @ W_aud + b_aud)
    ind = c @ W_ind
    cvec = aud[0] + ind
    geo = jnp.sum((l @ R) ** 2) * 1e-4
    Wh = W_rgb[:F]
    Wd = R @ W_rgb[F:]
    bc = b_rgb + cvec @ Wh
    ac = cvec[:3] + geo
    cflat = jnp.concatenate([
        Wh.reshape(-1), Wd.reshape(-1), bc, ac,
        jnp.zeros((_NCONST - 39,), jnp.float32)])
    csplat = jnp.repeat(cflat, L)

    planes = [feature_grid[:, j] for j in range(F)]
    ft_stream = _sc_interleave(*planes)
    ft8 = ft_stream.reshape(G3, F)

    xs = [x[:, i] for i in range(3)]
    ds_ = [d[:, i] for i in range(3)]
    sig, r, g, b, a0, a1, a2 = _sc_field(
        *xs, *ds_, density_grid, ft8, csplat)
    rgb = jnp.stack([r, g, b], axis=-1)
    amb = jnp.stack([a0, a1, a2], axis=-1)
    return sig, rgb, amb


# speedup vs baseline: 15.1460x; 2.1219x over previous
import functools
import jax
import jax.numpy as jnp
from jax import lax
from jax.experimental import pallas as pl
from jax.experimental.pallas import tpu as pltpu
from jax.experimental.pallas import tpu_sc as plsc

G = 128
F = 8
BOUND = 1.0
N = 2097152
G3 = G ** 3

NC, NS, L = 2, 16, 16
NW = NC * NS
M = N // NW
B = 2048
NBLK = M // B
CHUNK = 128
NCHUNK = B // CHUNK

MV = G3 // NW
BV = 2048
NBV = MV // BV

_ROW_WH = 0
_ROW_WD = 24
_ROW_BC = 33
_ROW_AC = 36
_NCONST = 40

_SC_PARAMS = pltpu.CompilerParams(
    use_tc_tiling_on_sc=False, needs_layout_passes=False)
_MESH = plsc.VectorSubcoreMesh(core_axis_name="c", subcore_axis_name="s")


_A_SCRATCH = (
    [pltpu.VMEM((BV,), jnp.float32) for _ in range(2 * F)]
    + [pltpu.VMEM((BV * F,), jnp.float32) for _ in range(2)]
    + [pltpu.SemaphoreType.DMA for _ in range(4)]
)


@functools.partial(
    pl.kernel,
    out_type=jax.ShapeDtypeStruct((G3 * F,), jnp.float32),
    mesh=_MESH,
    compiler_params=_SC_PARAMS,
    scratch_types=_A_SCRATCH,
)
def _sc_interleave(p0, p1, p2, p3, p4, p5, p6, p7, ft_h, *scr):
    planes = (p0, p1, p2, p3, p4, p5, p6, p7)
    bufs = (scr[0:F], scr[F:2 * F])
    ov = scr[2 * F:2 * F + 2]
    sem_i = scr[2 * F + 2:2 * F + 4]
    sem_o = scr[2 * F + 4:2 * F + 6]
    wid = lax.axis_index("s") * NC + lax.axis_index("c")
    iota = lax.iota(jnp.int32, L)

    def in_copies(v, p):
        base = wid * MV + v * BV
        return [pltpu.make_async_copy(
            planes[j].at[pl.ds(base, BV)], bufs[p][j], sem_i[p])
            for j in range(F)]

    def out_copy(v, p):
        base = wid * MV + v * BV
        return pltpu.make_async_copy(
            ov[p], ft_h.at[pl.ds(base * F, BV * F)], sem_o[p])

    for c in in_copies(0, 0):
        c.start()
    for v in range(NBV):
        p = v & 1
        for c in in_copies(v, p):
            c.wait()
        if v + 1 < NBV:
            for c in in_copies(v + 1, 1 - p):
                c.start()
        if v >= 2:
            out_copy(v - 2, p).wait()

        def ilv(g, carry):
            rows = (g * L + iota) * F
            for j in range(F):
                vj = bufs[p][j][pl.ds(g * L, L)]
                plsc.store_scatter(ov[p], [rows + j], vj)
            return carry

        lax.fori_loop(0, BV // L, ilv, 0)

        out_copy(v, p).start()
    out_copy(NBV - 2, 0).wait()
    out_copy(NBV - 1, 1).wait()


_B_SCRATCH = (
    [pltpu.VMEM((B,), jnp.float32) for _ in range(2 * 6)]
    + [pltpu.VMEM((B,), jnp.int32) for _ in range(2)]
    + [pltpu.VMEM((B, F), jnp.float32) for _ in range(2)]
    + [pltpu.VMEM((B,), jnp.float32) for _ in range(2)]
    + [pltpu.VMEM((B,), jnp.float32) for _ in range(2 * 7)]
    + [pltpu.VMEM((_NCONST * L,), jnp.float32)]
    + [pltpu.SemaphoreType.DMA for _ in range(8)]
)


@functools.partial(
    pl.kernel,
    out_type=tuple(jax.ShapeDtypeStruct((N,), jnp.float32) for _ in range(7)),
    mesh=_MESH,
    compiler_params=_SC_PARAMS,
    scratch_types=_B_SCRATCH,
)
def _sc_field(x0_h, x1_h, x2_h, d0_h, d1_h, d2_h, dens_h, ft_h, c_h,
              sig_h, r_h, g_h, b_h, a0_h, a1_h, a2_h, *scr):
    ins_h = (x0_h, x1_h, x2_h, d0_h, d1_h, d2_h)
    outs_h = (sig_h, r_h, g_h, b_h, a0_h, a1_h, a2_h)
    xv = (scr[0:6], scr[6:12])
    idxv = scr[12:14]
    featv = scr[14:16]
    densv = scr[16:18]
    outs = (scr[18:25], scr[25:32])
    cv = scr[32]
    sem_in = scr[33:35]
    sem_f = scr[35:37]
    sem_d = scr[37:39]
    sem_out = scr[39:41]

    wid = lax.axis_index("s") * NC + lax.axis_index("c")
    pltpu.sync_copy(c_h, cv)
    crows = [cv[pl.ds(r * L, L)] for r in range(39)]
    iota = lax.iota(jnp.int32, L)

    def base(b):
        return wid * M + b * B

    def in_copies(b, p):
        return [pltpu.make_async_copy(
            ins_h[k].at[pl.ds(base(b), B)], xv[p][k], sem_in[p])
            for k in range(6)]

    def gather_copies(p):
        cps = []
        for ci in range(NCHUNK):
            s = pl.ds(ci * CHUNK, CHUNK)
            cps.append(pltpu.make_async_copy(
                ft_h.at[idxv[p].at[s]], featv[p].at[s, :], sem_f[p]))
            cps.append(pltpu.make_async_copy(
                dens_h.at[idxv[p].at[s]], densv[p].at[s], sem_d[p]))
        return cps

    def out_copies(b, p):
        return [pltpu.make_async_copy(
            outs[p][k], outs_h[k].at[pl.ds(base(b), B)], sem_out[p])
            for k in range(7)]

    def vox(b, p):
        x0, x1, x2 = xv[p][0], xv[p][1], xv[p][2]

        def _vox(g, carry):
            s = pl.ds(g * L, L)

            def q(v):
                v = (v + BOUND) * ((G - 1) / (2.0 * BOUND))
                v = jnp.minimum(jnp.maximum(v, 0.0), float(G - 1))
                return v.astype(jnp.int32)

            idxv[p][s] = q(x0[s]) * (G * G) + q(x1[s]) * G + q(x2[s])
            return carry

        lax.fori_loop(0, B // L, _vox, 0)

    def comp(b, p):
        dv0, dv1, dv2 = xv[p][3], xv[p][4], xv[p][5]
        fv, dnv = featv[p], densv[p]
        ob = outs[p]

        def _comp(g, carry):
            s = pl.ds(g * L, L)
            rows = g * L + iota
            ob[0][s] = jnp.exp(dnv[s])
            f = [plsc.load_gather(fv, [rows, jnp.full((L,), j, jnp.int32)])
                 for j in range(F)]
            dd = (dv0[s], dv1[s], dv2[s])
            for kk in range(3):
                acc = crows[_ROW_BC + kk]
                for j in range(F):
                    acc = acc + f[j] * crows[_ROW_WH + j * 3 + kk]
                for i in range(3):
                    acc = acc + dd[i] * crows[_ROW_WD + i * 3 + kk]
                ob[1 + kk][s] = 1.0 / (1.0 + jnp.exp(-acc))
                ob[4 + kk][s] = f[kk] + crows[_ROW_AC + kk]
            return carry

        lax.fori_loop(0, B // L, _comp, 0)

    def emit(b, p, comp_prev, wait_out_prev, issue_in_next):
        q = 1 - p
        for c in in_copies(b, p):
            c.wait()
        vox(b, p)
        gd = gather_copies(p)
        for c in gd:
            c.start()
        if comp_prev:
            if wait_out_prev:
                for c in out_copies(b - 3, q):
                    c.wait()
            comp(b - 1, q)
            for c in out_copies(b - 1, q):
                c.start()
        if issue_in_next:
            for c in in_copies(b + 1, q):
                c.start()
        for c in gd:
            c.wait()

    for c in in_copies(0, 0):
        c.start()
    emit(0, 0, False, False, True)
    emit(1, 1, True, False, True)
    emit(2, 0, True, False, True)

    def body2(k, carry):
        b = 2 * k + 3
        emit(b, 1, True, True, True)
        emit(b + 1, 0, True, True, True)
        return carry

    lax.fori_loop(0, (NBLK - 4) // 2, body2, 0)

    emit(NBLK - 1, 1, True, True, False)
    for c in out_copies(NBLK - 3, 1):
        c.wait()
    comp(NBLK - 1, 1)
    for c in out_copies(NBLK - 1, 1):
        c.start()
    for c in out_copies(NBLK - 2, 0):
        c.wait()
    for c in out_copies(NBLK - 1, 1):
        c.wait()


def kernel(x, d, l, R, a, c, density_grid, feature_grid, W_aud, b_aud,
           W_ind, W_rgb, b_rgb):
    aud = jnp.tanh(a @ W_aud + b_aud)
    ind = c @ W_ind
    cvec = aud[0] + ind
    geo = jnp.sum((l @ R) ** 2) * 1e-4
    Wh = W_rgb[:F]
    Wd = R @ W_rgb[F:]
    bc = b_rgb + cvec @ Wh
    ac = cvec[:3] + geo
    cflat = jnp.concatenate([
        Wh.reshape(-1), Wd.reshape(-1), bc, ac,
        jnp.zeros((_NCONST - 39,), jnp.float32)])
    csplat = jnp.repeat(cflat, L)

    planes = [feature_grid[:, j] for j in range(F)]
    ft_stream = _sc_interleave(*planes)
    ft8 = ft_stream.reshape(G3, F)

    xs = [x[:, i] for i in range(3)]
    ds_ = [d[:, i] for i in range(3)]
    sig, r, g, b, a0, a1, a2 = _sc_field(
        *xs, *ds_, density_grid, ft8, csplat)
    rgb = jnp.stack([r, g, b], axis=-1)
    amb = jnp.stack([a0, a1, a2], axis=-1)
    return sig, rgb, amb


# speedup vs baseline: 15.3100x; 1.0108x over previous
import functools
import jax
import jax.numpy as jnp
from jax import lax
from jax.experimental import pallas as pl
from jax.experimental.pallas import tpu as pltpu
from jax.experimental.pallas import tpu_sc as plsc

G = 128
F = 8
BOUND = 1.0
N = 2097152
G3 = G ** 3

NC, NS, L = 2, 16, 16
NW = NC * NS
M = N // NW
B = 2048
NBLK = M // B
CHUNK = 2048
NCHUNK = B // CHUNK

MV = G3 // NW
BV = 2048
NBV = MV // BV

_ROW_WH = 0
_ROW_WD = 24
_ROW_BC = 33
_ROW_AC = 36
_NCONST = 40

_SC_PARAMS = pltpu.CompilerParams(
    use_tc_tiling_on_sc=False, needs_layout_passes=False)
_MESH = plsc.VectorSubcoreMesh(core_axis_name="c", subcore_axis_name="s")


R = 16
NPL = F + 1

_A_SCRATCH = (
    [pltpu.VMEM((BV,), jnp.float32) for _ in range(2 * NPL)]
    + [pltpu.VMEM((BV * R,), jnp.float32) for _ in range(2)]
    + [pltpu.SemaphoreType.DMA for _ in range(4)]
)


@functools.partial(
    pl.kernel,
    out_type=jax.ShapeDtypeStruct((G3 * R,), jnp.float32),
    mesh=_MESH,
    compiler_params=_SC_PARAMS,
    scratch_types=_A_SCRATCH,
)
def _sc_interleave(p0, p1, p2, p3, p4, p5, p6, p7, ft_h, *scr):
    planes = (p0, p1, p2, p3, p4, p5, p6, p7)
    bufs = (scr[0:NPL], scr[NPL:2 * NPL])
    ov = scr[2 * NPL:2 * NPL + 2]
    sem_i = scr[2 * NPL + 2:2 * NPL + 4]
    sem_o = scr[2 * NPL + 4:2 * NPL + 6]
    wid = lax.axis_index("s") * NC + lax.axis_index("c")
    iota = lax.iota(jnp.int32, L)

    def in_copies(v, p):
        base = wid * MV + v * BV
        return [pltpu.make_async_copy(
            planes[j].at[pl.ds(base, BV)], bufs[p][j], sem_i[p])
            for j in range(NPL)]

    def out_copy(v, p):
        base = wid * MV + v * BV
        return pltpu.make_async_copy(
            ov[p], ft_h.at[pl.ds(base * R, BV * R)], sem_o[p])

    for c in in_copies(0, 0):
        c.start()
    for v in range(NBV):
        p = v & 1
        for c in in_copies(v, p):
            c.wait()
        if v + 1 < NBV:
            for c in in_copies(v + 1, 1 - p):
                c.start()
        if v >= 2:
            out_copy(v - 2, p).wait()

        def ilv(g, carry):
            for u in range(2):
                g2 = g * 2 + u
                rows = (g2 * L + iota) * R
                for j in range(NPL):
                    vj = bufs[p][j][pl.ds(g2 * L, L)]
                    plsc.store_scatter(ov[p], [rows + j], vj)
            return carry

        lax.fori_loop(0, BV // L // 2, ilv, 0)

        out_copy(v, p).start()
    out_copy(NBV - 2, 0).wait()
    out_copy(NBV - 1, 1).wait()


_B_SCRATCH = (
    [pltpu.VMEM((B,), jnp.float32) for _ in range(2 * 6)]
    + [pltpu.VMEM((B,), jnp.int32) for _ in range(2)]
    + [pltpu.VMEM((B, R), jnp.float32) for _ in range(2)]
    + [pltpu.VMEM((B,), jnp.float32) for _ in range(2 * 7)]
    + [pltpu.VMEM((_NCONST * L,), jnp.float32)]
    + [pltpu.SemaphoreType.DMA for _ in range(6)]
)


@functools.partial(
    pl.kernel,
    out_type=tuple(jax.ShapeDtypeStruct((N,), jnp.float32) for _ in range(7)),
    mesh=_MESH,
    compiler_params=_SC_PARAMS,
    scratch_types=_B_SCRATCH,
)
def _sc_field(x0_h, x1_h, x2_h, d0_h, d1_h, d2_h, dens_h, ft_h, c_h,
              sig_h, r_h, g_h, b_h, a0_h, a1_h, a2_h, *scr):
    ins_h = (x0_h, x1_h, x2_h, d0_h, d1_h, d2_h)
    outs_h = (sig_h, r_h, g_h, b_h, a0_h, a1_h, a2_h)
    xv = (scr[0:6], scr[6:12])
    idxv = scr[12:14]
    featv = scr[14:16]
    densv = scr[16:18]
    outs = (scr[18:25], scr[25:32])
    cv = scr[32]
    sem_in = scr[33:35]
    sem_f = scr[35:37]
    sem_d = scr[37:39]
    sem_out = scr[39:41]

    wid = lax.axis_index("s") * NC + lax.axis_index("c")
    pltpu.sync_copy(c_h, cv)
    crows = [cv[pl.ds(r * L, L)] for r in range(39)]
    iota = lax.iota(jnp.int32, L)

    def base(b):
        return wid * M + b * B

    def in_copies(b, p):
        return [pltpu.make_async_copy(
            ins_h[k].at[pl.ds(base(b), B)], xv[p][k], sem_in[p])
            for k in range(6)]

    def gather_copies(p):
        cps = []
        for ci in range(NCHUNK):
            s = pl.ds(ci * CHUNK, CHUNK)
            cps.append(pltpu.make_async_copy(
                ft_h.at[idxv[p].at[s]], featv[p].at[s, :], sem_f[p]))
            cps.append(pltpu.make_async_copy(
                dens_h.at[idxv[p].at[s]], densv[p].at[s], sem_d[p]))
        return cps

    def out_copies(b, p):
        return [pltpu.make_async_copy(
            outs[p][k], outs_h[k].at[pl.ds(base(b), B)], sem_out[p])
            for k in range(7)]

    def vox(b, p):
        x0, x1, x2 = xv[p][0], xv[p][1], xv[p][2]

        def _vox(g, carry):
            s = pl.ds(g * L, L)

            def q(v):
                v = (v + BOUND) * ((G - 1) / (2.0 * BOUND))
                v = jnp.minimum(jnp.maximum(v, 0.0), float(G - 1))
                return v.astype(jnp.int32)

            idxv[p][s] = q(x0[s]) * (G * G) + q(x1[s]) * G + q(x2[s])
            return carry

        def _vox4(g, carry):
            for u in range(4):
                _vox(g * 4 + u, carry)
            return carry

        lax.fori_loop(0, B // L // 4, _vox4, 0)

    def comp(b, p):
        dv0, dv1, dv2 = xv[p][3], xv[p][4], xv[p][5]
        fv, dnv = featv[p], densv[p]
        ob = outs[p]

        def _comp(g, carry):
            s = pl.ds(g * L, L)
            rows = g * L + iota
            ob[0][s] = jnp.exp(dnv[s])
            f = [plsc.load_gather(fv, [rows, jnp.full((L,), j, jnp.int32)])
                 for j in range(F)]
            dd = (dv0[s], dv1[s], dv2[s])
            for kk in range(3):
                acc = crows[_ROW_BC + kk]
                for j in range(F):
                    acc = acc + f[j] * crows[_ROW_WH + j * 3 + kk]
                for i in range(3):
                    acc = acc + dd[i] * crows[_ROW_WD + i * 3 + kk]
                ob[1 + kk][s] = 1.0 / (1.0 + jnp.exp(-acc))
                ob[4 + kk][s] = f[kk] + crows[_ROW_AC + kk]
            return carry

        lax.fori_loop(0, B // L, _comp, 0)

    def emit(b, p, comp_prev, wait_out_prev, issue_in_next):
        q = 1 - p
        for c in in_copies(b, p):
            c.wait()
        vox(b, p)
        gd = gather_copies(p)
        for c in gd:
            c.start()
        if comp_prev:
            if wait_out_prev:
                for c in out_copies(b - 3, q):
                    c.wait()
            comp(b - 1, q)
            for c in out_copies(b - 1, q):
                c.start()
        if issue_in_next:
            for c in in_copies(b + 1, q):
                c.start()
        for c in gd:
            c.wait()

    for c in in_copies(0, 0):
        c.start()
    emit(0, 0, False, False, True)
    emit(1, 1, True, False, True)
    emit(2, 0, True, False, True)

    def body2(k, carry):
        b = 2 * k + 3
        emit(b, 1, True, True, True)
        emit(b + 1, 0, True, True, True)
        return carry

    lax.fori_loop(0, (NBLK - 4) // 2, body2, 0)

    emit(NBLK - 1, 1, True, True, False)
    for c in out_copies(NBLK - 3, 1):
        c.wait()
    comp(NBLK - 1, 1)
    for c in out_copies(NBLK - 1, 1):
        c.start()
    for c in out_copies(NBLK - 2, 0):
        c.wait()
    for c in out_copies(NBLK - 1, 1):
        c.wait()


def kernel(x, d, l, R, a, c, density_grid, feature_grid, W_aud, b_aud,
           W_ind, W_rgb, b_rgb):
    aud = jnp.tanh(a @ W_aud + b_aud)
    ind = c @ W_ind
    cvec = aud[0] + ind
    geo = jnp.sum((l @ R) ** 2) * 1e-4
    Wh = W_rgb[:F]
    Wd = R @ W_rgb[F:]
    bc = b_rgb + cvec @ Wh
    ac = cvec[:3] + geo
    cflat = jnp.concatenate([
        Wh.reshape(-1), Wd.reshape(-1), bc, ac,
        jnp.zeros((_NCONST - 39,), jnp.float32)])
    csplat = jnp.repeat(cflat, L)

    planes = [feature_grid[:, j] for j in range(F)]
    ft_stream = _sc_interleave(density_grid, *planes)
    ft16 = ft_stream.reshape(G3, R)

    xs = [x[:, i] for i in range(3)]
    ds_ = [d[:, i] for i in range(3)]
    sig, r, g, b, a0, a1, a2 = _sc_field(
        *xs, *ds_, ft16, csplat)
    rgb = jnp.stack([r, g, b], axis=-1)
    amb = jnp.stack([a0, a1, a2], axis=-1)
    return sig, rgb, amb
